# baseline (speedup 1.0000x reference)
import functools

import jax
import jax.numpy as jnp
from jax import lax
from jax.experimental import pallas as pl
from jax.experimental.pallas import tpu as pltpu
from jax.experimental.pallas import tpu_sc as plsc

_GAMMA = 12.0
_DIM = 64
_BATCH = 16384
_NUM = 1000000
_NC = 2
_NS = 16
_NW = _NC * _NS
_BPW = _BATCH // _NW
_NCHUNK = 4
_CHUNK = _BPW // _NCHUNK
_LANES = 16

_TCOLS = 16384


def _tp_body(in_ref, out_ref):
    eye = jnp.eye(_DIM, dtype=jnp.float32)
    out_ref[...] = jax.lax.dot_general(
        in_ref[...], eye, (((0,), (0,)), ((), ())),
        preferred_element_type=jnp.float32)


def _to_row_major(tableT):
    n = tableT.shape[1]
    grid = (n + _TCOLS - 1) // _TCOLS
    return pl.pallas_call(
        _tp_body,
        grid=(grid,),
        in_specs=[pl.BlockSpec((_DIM, _TCOLS), lambda c: (0, c))],
        out_specs=pl.BlockSpec((_TCOLS, _DIM), lambda c: (c, 0)),
        out_shape=jax.ShapeDtypeStruct((n, _DIM), jnp.float32),
    )(tableT)


def _body(head_hbm, ridx_hbm, ent_hbm, rel_hbm, out_hbm,
          hidx_v, ridx_v, hbuf, rbuf, tbuf, out_v, sem):
    wid = lax.axis_index("s") * _NC + lax.axis_index("c")
    pltpu.sync_copy(head_hbm.at[pl.ds(wid * _NCHUNK, _NCHUNK)], hidx_v)
    pltpu.sync_copy(ridx_hbm.at[pl.ds(wid * _NCHUNK, _NCHUNK)], ridx_v)

    copies = []
    for j in range(_NCHUNK):
        sl = pl.ds(j * _CHUNK, _CHUNK)
        copies.append(pltpu.async_copy(ent_hbm.at[hidx_v.at[j]], hbuf.at[sl], sem))
        copies.append(pltpu.async_copy(rel_hbm.at[ridx_v.at[j]], rbuf.at[sl], sem))
        copies.append(pltpu.async_copy(ent_hbm.at[ridx_v.at[j]], tbuf.at[sl], sem))
    for c in copies:
        c.wait()

    lanes = lax.broadcasted_iota(jnp.int32, (_LANES,), 0)

    def group(g, carry):
        rows = g * _LANES + lanes
        acc = jnp.zeros((_LANES,), jnp.float32)
        for d in range(_DIM):
            dvec = jnp.full((_LANES,), d, jnp.int32)
            h = plsc.load_gather(hbuf, [rows, dvec])
            r = plsc.load_gather(rbuf, [rows, dvec])
            t = plsc.load_gather(tbuf, [rows, dvec])
            acc = acc + jnp.abs(h + r - t)
        out_v[pl.ds(g * _LANES, _LANES)] = 1.0 / (1.0 + jnp.exp(acc - _GAMMA))
        return carry

    lax.fori_loop(0, _BPW // _LANES, group, 0)
    pltpu.sync_copy(out_v, out_hbm.at[pl.ds(wid * _BPW, _BPW)])


_transe_sc = functools.partial(
    pl.kernel,
    out_type=jax.ShapeDtypeStruct((_BATCH,), jnp.float32),
    mesh=plsc.VectorSubcoreMesh(core_axis_name="c", subcore_axis_name="s"),
    scratch_types=[
        pltpu.VMEM((_NCHUNK, _CHUNK), jnp.int32),
        pltpu.VMEM((_NCHUNK, _CHUNK), jnp.int32),
        pltpu.VMEM((_BPW, _DIM), jnp.float32),
        pltpu.VMEM((_BPW, _DIM), jnp.float32),
        pltpu.VMEM((_BPW, _DIM), jnp.float32),
        pltpu.VMEM((_BPW,), jnp.float32),
        pltpu.SemaphoreType.DMA,
    ],
    compiler_params=pltpu.CompilerParams(
        needs_layout_passes=False, use_tc_tiling_on_sc=False),
)(_body)


def kernel(x, emb_ent_real, emb_rel_real):
    head = x[:, 0].astype(jnp.int32).reshape(_NW * _NCHUNK, _CHUNK)
    ridx = x[:, 1].astype(jnp.int32).reshape(_NW * _NCHUNK, _CHUNK)
    ent_rm = _to_row_major(emb_ent_real.T)
    rel_rm = _to_row_major(emb_rel_real.T)
    return _transe_sc(head, ridx, ent_rm, rel_rm)


# speedup vs baseline: 2.3424x; 2.3424x over previous
import functools

import jax
import jax.numpy as jnp
from jax import lax
from jax.experimental import pallas as pl
from jax.experimental.pallas import tpu as pltpu
from jax.experimental.pallas import tpu_sc as plsc

_GAMMA = 12.0
_DIM = 64
_BATCH = 16384
_NUM = 1000000
_NC = 2
_NS = 16
_NW = _NC * _NS
_BPW = _BATCH // _NW
_EHITS = 2 * _BATCH // _NW
_RHITS = _BATCH // _NW
_LANES = 16
_W = 640
_MAXSTART = 999296
_TAIL = _NUM - 64
_RING = 8
_NSLOT = 3 * _BATCH


def _scan_body(sev, sep, srv, srp, entT, relT, tle, tlr, scratch_out,
               ev_v, ep_v, rv_v, rp_v, bbuf, tail_v, ering, sem):
    wid = lax.axis_index("s") * _NC + lax.axis_index("c")
    pltpu.sync_copy(sev.at[wid], ev_v)
    pltpu.sync_copy(sep.at[wid], ep_v)
    pltpu.sync_copy(srv.at[wid], rv_v)
    pltpu.sync_copy(srp.at[wid], rp_v)
    lanes = lax.broadcasted_iota(jnp.int32, (_LANES,), 0)
    zero16 = jnp.zeros((_LANES,), jnp.int32)

    def stream(tab, tailtab, vals_v, pos_v, nhits):
        pltpu.sync_copy(tailtab, tail_v)

        def one_hit(i, k, cur):
            v = vals_v[pl.ds(i, _LANES)][0]
            p = pos_v[pl.ds(i, _LANES)][0]
            start = jnp.minimum((v >> 9) * 512, _MAXSTART)

            @pl.when(start != cur)
            def _():
                pltpu.sync_copy(tab.at[:, pl.ds(start, _W)], bbuf)

            colm = zero16 + jnp.minimum(v - start, _W - 1)
            colt = zero16 + jnp.maximum(v - _TAIL, 0)
            istail = v >= _TAIL
            for c in range(_DIM // _LANES):
                dvec = c * _LANES + lanes
                mv = plsc.load_gather(bbuf, [dvec, colm])
                tv = plsc.load_gather(tail_v, [dvec, colt])
                ering[k, pl.ds(c * _LANES, _LANES)] = jnp.where(istail, tv, mv)
            pltpu.async_copy(ering.at[k],
                             scratch_out.at[pl.ds(p * _DIM, _DIM)], sem)
            return start

        def ring_group(g, cur):
            for j in range(_RING):
                cur = one_hit(g * _RING + j, j, cur)
            for j in range(_RING):
                pltpu.make_async_copy(ering.at[0],
                                      scratch_out.at[pl.ds(0, _DIM)],
                                      sem).wait()
            return cur

        lax.fori_loop(0, nhits // _RING, ring_group, jnp.int32(-1))

    stream(entT, tle, ev_v, ep_v, _EHITS)
    stream(relT, tlr, rv_v, rp_v, _RHITS)


_scan = functools.partial(
    pl.kernel,
    out_type=jax.ShapeDtypeStruct((_NSLOT * _DIM,), jnp.float32),
    mesh=plsc.VectorSubcoreMesh(core_axis_name="c", subcore_axis_name="s"),
    scratch_types=[
        pltpu.VMEM((_EHITS + _LANES,), jnp.int32),
        pltpu.VMEM((_EHITS + _LANES,), jnp.int32),
        pltpu.VMEM((_RHITS + _LANES,), jnp.int32),
        pltpu.VMEM((_RHITS + _LANES,), jnp.int32),
        pltpu.VMEM((_DIM, _W), jnp.float32),
        pltpu.VMEM((_DIM, 64), jnp.float32),
        pltpu.VMEM((_RING, _DIM), jnp.float32),
        pltpu.SemaphoreType.DMA,
    ],
    compiler_params=pltpu.CompilerParams(
        needs_layout_passes=False, use_tc_tiling_on_sc=True),
)(_scan_body)


def _compute_body(scr, out_hbm, hbuf, rbuf, tbuf, out_v, sem):
    wid = lax.axis_index("s") * _NC + lax.axis_index("c")
    base = wid * _BPW
    c1 = pltpu.async_copy(scr.at[pl.ds(base, _BPW)], hbuf, sem)
    c2 = pltpu.async_copy(scr.at[pl.ds(_BATCH + base, _BPW)], tbuf, sem)
    c3 = pltpu.async_copy(scr.at[pl.ds(2 * _BATCH + base, _BPW)], rbuf, sem)
    c1.wait()
    c2.wait()
    c3.wait()
    lanes = lax.broadcasted_iota(jnp.int32, (_LANES,), 0)

    def group(g, carry):
        rows = g * _LANES + lanes
        acc = jnp.zeros((_LANES,), jnp.float32)
        for d in range(_DIM):
            dvec = jnp.full((_LANES,), d, jnp.int32)
            h = plsc.load_gather(hbuf, [rows, dvec])
            r = plsc.load_gather(rbuf, [rows, dvec])
            t = plsc.load_gather(tbuf, [rows, dvec])
            acc = acc + jnp.abs(h + r - t)
        out_v[pl.ds(g * _LANES, _LANES)] = 1.0 / (1.0 + jnp.exp(acc - _GAMMA))
        return carry

    lax.fori_loop(0, _BPW // _LANES, group, 0)
    pltpu.sync_copy(out_v, out_hbm.at[pl.ds(base, _BPW)])


_compute = functools.partial(
    pl.kernel,
    out_type=jax.ShapeDtypeStruct((_BATCH,), jnp.float32),
    mesh=plsc.VectorSubcoreMesh(core_axis_name="c", subcore_axis_name="s"),
    scratch_types=[
        pltpu.VMEM((_BPW, _DIM), jnp.float32),
        pltpu.VMEM((_BPW, _DIM), jnp.float32),
        pltpu.VMEM((_BPW, _DIM), jnp.float32),
        pltpu.VMEM((_BPW,), jnp.float32),
        pltpu.SemaphoreType.DMA,
    ],
    compiler_params=pltpu.CompilerParams(
        needs_layout_passes=False, use_tc_tiling_on_sc=False),
)(_compute_body)


def _pad16(a):
    return jnp.pad(a, ((0, 0), (0, _LANES)))


def kernel(x, emb_ent_real, emb_rel_real):
    hv = x[:, 0].astype(jnp.int32)
    rv = x[:, 1].astype(jnp.int32)
    ev = jnp.concatenate([hv, rv])
    eo = jnp.argsort(ev).astype(jnp.int32)
    ro = jnp.argsort(rv).astype(jnp.int32)
    sev = _pad16(ev[eo].reshape(_NW, _EHITS))
    sep = _pad16(eo.reshape(_NW, _EHITS))
    srv = _pad16(rv[ro].reshape(_NW, _RHITS))
    srp = _pad16((ro + 2 * _BATCH).reshape(_NW, _RHITS))
    scratch = _scan(sev, sep, srv, srp,
                    emb_ent_real.T, emb_rel_real.T,
                    emb_ent_real[_TAIL:, :].T, emb_rel_real[_TAIL:, :].T)
    return _compute(scratch.reshape(_NSLOT, _DIM))


# speedup vs baseline: 3.4677x; 1.4804x over previous
import functools

import jax
import jax.numpy as jnp
from jax import lax
from jax.experimental import pallas as pl
from jax.experimental.pallas import tpu as pltpu
from jax.experimental.pallas import tpu_sc as plsc

_GAMMA = 12.0
_DIM = 64
_BATCH = 16384
_NUM = 1000000
_NC = 2
_NS = 16
_NW = _NC * _NS
_BPW = _BATCH // _NW
_EHITS = 2 * _BATCH // _NW
_RHITS = _BATCH // _NW
_LANES = 16
_W = 640
_NBLK = _NUM // _W
_TAIL = _NBLK * _W
_NTAIL = _NUM - _TAIL
_RING = 8
_NSLOT = 3 * _BATCH


def _scan_body(sev, sep, srv, srp, entT, relT, tle, tlr, scratch_out,
               ev_v, ep_v, rv_v, rp_v, bbuf, tail_v, ering, sem, psem):
    wid = lax.axis_index("s") * _NC + lax.axis_index("c")
    pltpu.sync_copy(sev.at[wid], ev_v)
    pltpu.sync_copy(sep.at[wid], ep_v)
    pltpu.sync_copy(srv.at[wid], rv_v)
    pltpu.sync_copy(srp.at[wid], rp_v)
    lanes = lax.broadcasted_iota(jnp.int32, (_LANES,), 0)
    zero16 = jnp.zeros((_LANES,), jnp.int32)

    def wait_pref(tab):
        pltpu.make_async_copy(tab.at[:, pl.ds(0, _W)],
                              bbuf.at[0], psem).wait()

    def stream(tab, tailtab, vals_v, pos_v, nhits):
        pltpu.sync_copy(tailtab, tail_v)

        def one_hit(i, k, state):
            cur, pref = state
            v = vals_v[pl.ds(i, _LANES)][0]
            p = pos_v[pl.ds(i, _LANES)][0]
            istail = v >= _TAIL
            blk = jnp.where(istail, cur, v // _W)

            @pl.when(blk != cur)
            def _():
                @pl.when(pref >= 0)
                def _():
                    wait_pref(tab)

                @pl.when(pref != blk)
                def _():
                    pltpu.sync_copy(tab.at[:, pl.ds(blk * _W, _W)],
                                    bbuf.at[blk % 2])

                @pl.when(blk + 1 < _NBLK)
                def _():
                    pltpu.async_copy(tab.at[:, pl.ds((blk + 1) * _W, _W)],
                                     bbuf.at[(blk + 1) % 2], psem)

            ncur = jnp.where(blk != cur, blk, cur)
            npref = jnp.where(blk != cur,
                              jnp.where(blk + 1 < _NBLK, blk + 1, -1),
                              pref)
            par16 = zero16 + ncur % 2
            colm = zero16 + jnp.minimum(v - ncur * _W, _W - 1)
            colt = zero16 + jnp.maximum(v - _TAIL, 0)
            for c in range(_DIM // _LANES):
                dvec = c * _LANES + lanes
                mv = plsc.load_gather(bbuf, [par16, dvec, colm])
                tv = plsc.load_gather(tail_v, [dvec, colt])
                ering[k, pl.ds(c * _LANES, _LANES)] = jnp.where(istail, tv, mv)
            pltpu.async_copy(ering.at[k],
                             scratch_out.at[pl.ds(p * _DIM, _DIM)], sem)
            return ncur, npref

        def ring_group(g, state):
            for j in range(_RING):
                state = one_hit(g * _RING + j, j, state)
            for j in range(_RING):
                pltpu.make_async_copy(ering.at[0],
                                      scratch_out.at[pl.ds(0, _DIM)],
                                      sem).wait()
            return state

        _, pref = lax.fori_loop(0, nhits // _RING, ring_group,
                                (jnp.int32(-1), jnp.int32(-1)))

        @pl.when(pref >= 0)
        def _():
            wait_pref(tab)

    stream(entT, tle, ev_v, ep_v, _EHITS)
    stream(relT, tlr, rv_v, rp_v, _RHITS)


_scan = functools.partial(
    pl.kernel,
    out_type=jax.ShapeDtypeStruct((_NSLOT * _DIM,), jnp.float32),
    mesh=plsc.VectorSubcoreMesh(core_axis_name="c", subcore_axis_name="s"),
    scratch_types=[
        pltpu.VMEM((_EHITS + _LANES,), jnp.int32),
        pltpu.VMEM((_EHITS + _LANES,), jnp.int32),
        pltpu.VMEM((_RHITS + _LANES,), jnp.int32),
        pltpu.VMEM((_RHITS + _LANES,), jnp.int32),
        pltpu.VMEM((2, _DIM, _W), jnp.float32),
        pltpu.VMEM((_DIM, _NTAIL), jnp.float32),
        pltpu.VMEM((_RING, _DIM), jnp.float32),
        pltpu.SemaphoreType.DMA,
        pltpu.SemaphoreType.DMA,
    ],
    compiler_params=pltpu.CompilerParams(
        needs_layout_passes=False, use_tc_tiling_on_sc=True),
)(_scan_body)


def _compute_body(scr, out_hbm, hbuf, rbuf, tbuf, out_v, sem):
    wid = lax.axis_index("s") * _NC + lax.axis_index("c")
    base = wid * _BPW
    c1 = pltpu.async_copy(scr.at[pl.ds(base, _BPW)], hbuf, sem)
    c2 = pltpu.async_copy(scr.at[pl.ds(_BATCH + base, _BPW)], tbuf, sem)
    c3 = pltpu.async_copy(scr.at[pl.ds(2 * _BATCH + base, _BPW)], rbuf, sem)
    c1.wait()
    c2.wait()
    c3.wait()
    lanes = lax.broadcasted_iota(jnp.int32, (_LANES,), 0)

    def group(g, carry):
        rows = g * _LANES + lanes
        acc = jnp.zeros((_LANES,), jnp.float32)
        for d in range(_DIM):
            dvec = jnp.full((_LANES,), d, jnp.int32)
            h = plsc.load_gather(hbuf, [rows, dvec])
            r = plsc.load_gather(rbuf, [rows, dvec])
            t = plsc.load_gather(tbuf, [rows, dvec])
            acc = acc + jnp.abs(h + r - t)
        out_v[pl.ds(g * _LANES, _LANES)] = 1.0 / (1.0 + jnp.exp(acc - _GAMMA))
        return carry

    lax.fori_loop(0, _BPW // _LANES, group, 0)
    pltpu.sync_copy(out_v, out_hbm.at[pl.ds(base, _BPW)])


_compute = functools.partial(
    pl.kernel,
    out_type=jax.ShapeDtypeStruct((_BATCH,), jnp.float32),
    mesh=plsc.VectorSubcoreMesh(core_axis_name="c", subcore_axis_name="s"),
    scratch_types=[
        pltpu.VMEM((_BPW, _DIM), jnp.float32),
        pltpu.VMEM((_BPW, _DIM), jnp.float32),
        pltpu.VMEM((_BPW, _DIM), jnp.float32),
        pltpu.VMEM((_BPW,), jnp.float32),
        pltpu.SemaphoreType.DMA,
    ],
    compiler_params=pltpu.CompilerParams(
        needs_layout_passes=False, use_tc_tiling_on_sc=False),
)(_compute_body)


def _pad16(a):
    return jnp.pad(a, ((0, 0), (0, _LANES)))


def kernel(x, emb_ent_real, emb_rel_real):
    hv = x[:, 0].astype(jnp.int32)
    rv = x[:, 1].astype(jnp.int32)
    ev = jnp.concatenate([hv, rv])
    eo = jnp.argsort(ev).astype(jnp.int32)
    ro = jnp.argsort(rv).astype(jnp.int32)
    sev = _pad16(ev[eo].reshape(_NW, _EHITS))
    sep = _pad16(eo.reshape(_NW, _EHITS))
    srv = _pad16(rv[ro].reshape(_NW, _RHITS))
    srp = _pad16((ro + 2 * _BATCH).reshape(_NW, _RHITS))
    scratch = _scan(sev, sep, srv, srp,
                    emb_ent_real.T, emb_rel_real.T,
                    emb_ent_real[_TAIL:, :].T, emb_rel_real[_TAIL:, :].T)
    return _compute(scratch.reshape(_NSLOT, _DIM))


# speedup vs baseline: 3.6632x; 1.0564x over previous
import functools

import jax
import jax.numpy as jnp
from jax import lax
from jax.experimental import pallas as pl
from jax.experimental.pallas import tpu as pltpu
from jax.experimental.pallas import tpu_sc as plsc

_GAMMA = 12.0
_DIM = 64
_BATCH = 16384
_NUM = 1000000
_NC = 2
_NS = 16
_NW = _NC * _NS
_BPW = _BATCH // _NW
_EHITS = 2 * _BATCH // _NW
_RHITS = _BATCH // _NW
_LANES = 16
_W = 640
_NBLK = _NUM // _W
_TAIL = _NBLK * _W
_NTAIL = _NUM - _TAIL
_RING = 8
_NSLOT = 3 * _BATCH


def _scan_body(sev, sep, srv, srp, entT, relT, tle, tlr, scratch_out,
               ev_v, ep_v, rv_v, rp_v, bbuf, tail_v, ering, sem, psem):
    wid = lax.axis_index("s") * _NC + lax.axis_index("c")
    pltpu.sync_copy(sev.at[wid], ev_v)
    pltpu.sync_copy(sep.at[wid], ep_v)
    pltpu.sync_copy(srv.at[wid], rv_v)
    pltpu.sync_copy(srp.at[wid], rp_v)
    lanes = lax.broadcasted_iota(jnp.int32, (_LANES,), 0)
    zero16 = jnp.zeros((_LANES,), jnp.int32)

    def wait_pref(tab):
        pltpu.make_async_copy(tab.at[:, pl.ds(0, _W)],
                              bbuf.at[0], psem).wait()

    def stream(tab, tailtab, vals_v, pos_v, nhits):
        pltpu.sync_copy(tailtab, tail_v)

        def one_hit(i, k, state):
            cur, pref = state
            v = vals_v[pl.ds(i, _LANES)][0]
            p = pos_v[pl.ds(i, _LANES)][0]
            istail = v >= _TAIL
            blk = jnp.where(istail, cur, v // _W)

            @pl.when(blk != cur)
            def _():
                @pl.when(pref >= 0)
                def _():
                    wait_pref(tab)

                @pl.when(pref != blk)
                def _():
                    pltpu.sync_copy(tab.at[:, pl.ds(blk * _W, _W)],
                                    bbuf.at[blk % 2])

                @pl.when(blk + 1 < _NBLK)
                def _():
                    pltpu.async_copy(tab.at[:, pl.ds((blk + 1) * _W, _W)],
                                     bbuf.at[(blk + 1) % 2], psem)

            ncur = jnp.where(blk != cur, blk, cur)
            npref = jnp.where(blk != cur,
                              jnp.where(blk + 1 < _NBLK, blk + 1, -1),
                              pref)
            par16 = zero16 + ncur % 2
            colm = zero16 + jnp.minimum(v - ncur * _W, _W - 1)
            colt = zero16 + jnp.maximum(v - _TAIL, 0)
            for c in range(_DIM // _LANES):
                dvec = c * _LANES + lanes
                mv = plsc.load_gather(bbuf, [par16, dvec, colm])
                tv = plsc.load_gather(tail_v, [dvec, colt])
                ering[k, pl.ds(c * _LANES, _LANES)] = jnp.where(istail, tv, mv)
            pltpu.async_copy(ering.at[k],
                             scratch_out.at[pl.ds(p * _DIM, _DIM)], sem)
            return ncur, npref

        def ring_group(g, state):
            for j in range(_RING):
                state = one_hit(g * _RING + j, j, state)
            for j in range(_RING):
                pltpu.make_async_copy(ering.at[0],
                                      scratch_out.at[pl.ds(0, _DIM)],
                                      sem).wait()
            return state

        _, pref = lax.fori_loop(0, nhits // _RING, ring_group,
                                (jnp.int32(-1), jnp.int32(-1)))

        @pl.when(pref >= 0)
        def _():
            wait_pref(tab)

    stream(entT, tle, ev_v, ep_v, _EHITS)
    stream(relT, tlr, rv_v, rp_v, _RHITS)


_scan = functools.partial(
    pl.kernel,
    out_type=jax.ShapeDtypeStruct((_NSLOT * _DIM,), jnp.float32),
    mesh=plsc.VectorSubcoreMesh(core_axis_name="c", subcore_axis_name="s"),
    scratch_types=[
        pltpu.VMEM((_EHITS + _LANES,), jnp.int32),
        pltpu.VMEM((_EHITS + _LANES,), jnp.int32),
        pltpu.VMEM((_RHITS + _LANES,), jnp.int32),
        pltpu.VMEM((_RHITS + _LANES,), jnp.int32),
        pltpu.VMEM((2, _DIM, _W), jnp.float32),
        pltpu.VMEM((_DIM, _NTAIL), jnp.float32),
        pltpu.VMEM((_RING, _DIM), jnp.float32),
        pltpu.SemaphoreType.DMA,
        pltpu.SemaphoreType.DMA,
    ],
    compiler_params=pltpu.CompilerParams(
        needs_layout_passes=False, use_tc_tiling_on_sc=True),
)(_scan_body)


def _compute_body(scr, out_hbm, hbuf, rbuf, tbuf, out_v, sem):
    wid = lax.axis_index("s") * _NC + lax.axis_index("c")
    base = wid * _BPW
    c1 = pltpu.async_copy(scr.at[pl.ds(base, _BPW)], hbuf, sem)
    c2 = pltpu.async_copy(scr.at[pl.ds(_BATCH + base, _BPW)], tbuf, sem)
    c3 = pltpu.async_copy(scr.at[pl.ds(2 * _BATCH + base, _BPW)], rbuf, sem)
    c1.wait()
    c2.wait()
    c3.wait()
    lanes = lax.broadcasted_iota(jnp.int32, (_LANES,), 0)

    zero16 = jnp.zeros((_LANES,), jnp.int32)

    def group(g, carry):
        rows = g * _LANES + lanes

        def dim_body(d, acc):
            dvec = zero16 + d
            h = plsc.load_gather(hbuf, [rows, dvec])
            r = plsc.load_gather(rbuf, [rows, dvec])
            t = plsc.load_gather(tbuf, [rows, dvec])
            return acc + jnp.abs(h + r - t)

        acc = lax.fori_loop(0, _DIM, dim_body, jnp.zeros((_LANES,), jnp.float32))
        out_v[pl.ds(g * _LANES, _LANES)] = 1.0 / (1.0 + jnp.exp(acc - _GAMMA))
        return carry

    lax.fori_loop(0, _BPW // _LANES, group, 0)
    pltpu.sync_copy(out_v, out_hbm.at[pl.ds(base, _BPW)])


_compute = functools.partial(
    pl.kernel,
    out_type=jax.ShapeDtypeStruct((_BATCH,), jnp.float32),
    mesh=plsc.VectorSubcoreMesh(core_axis_name="c", subcore_axis_name="s"),
    scratch_types=[
        pltpu.VMEM((_BPW, _DIM), jnp.float32),
        pltpu.VMEM((_BPW, _DIM), jnp.float32),
        pltpu.VMEM((_BPW, _DIM), jnp.float32),
        pltpu.VMEM((_BPW,), jnp.float32),
        pltpu.SemaphoreType.DMA,
    ],
    compiler_params=pltpu.CompilerParams(
        needs_layout_passes=False, use_tc_tiling_on_sc=False),
)(_compute_body)


def _pad16(a):
    return jnp.pad(a, ((0, 0), (0, _LANES)))


def kernel(x, emb_ent_real, emb_rel_real):
    hv = x[:, 0].astype(jnp.int32)
    rv = x[:, 1].astype(jnp.int32)
    ev = jnp.concatenate([hv, rv])
    evs, eo = lax.sort((ev, jnp.arange(2 * _BATCH, dtype=jnp.int32)),
                       num_keys=1)
    rvs, ro = lax.sort((rv, jnp.arange(2 * _BATCH, 3 * _BATCH,
                                       dtype=jnp.int32)), num_keys=1)
    sev = _pad16(evs.reshape(_NW, _EHITS))
    sep = _pad16(eo.reshape(_NW, _EHITS))
    srv = _pad16(rvs.reshape(_NW, _RHITS))
    srp = _pad16(ro.reshape(_NW, _RHITS))
    scratch = _scan(sev, sep, srv, srp,
                    emb_ent_real.T, emb_rel_real.T,
                    emb_ent_real[_TAIL:, :].T, emb_rel_real[_TAIL:, :].T)
    return _compute(scratch.reshape(_NSLOT, _DIM))


# speedup vs baseline: 3.9071x; 1.0666x over previous
import functools

import jax
import jax.numpy as jnp
from jax import lax
from jax.experimental import pallas as pl
from jax.experimental.pallas import tpu as pltpu
from jax.experimental.pallas import tpu_sc as plsc

_GAMMA = 12.0
_DIM = 64
_BATCH = 16384
_NUM = 1000000
_NC = 2
_NS = 16
_NW = _NC * _NS
_BPW = _BATCH // _NW
_EHITS = 2 * _BATCH // _NW
_RHITS = _BATCH // _NW
_LANES = 16
_W = 768
_NBLK = _NUM // _W
_TAIL = _NBLK * _W
_NTAIL = _NUM - _TAIL
_RING = 8
_NSLOT = 3 * _BATCH


def _scan_body(sev, sep, srv, srp, entT, relT, tle, tlr, scratch_out,
               ev_v, ep_v, rv_v, rp_v, bbuf, tail_v, ering, sem, psem):
    wid = lax.axis_index("s") * _NC + lax.axis_index("c")
    pltpu.sync_copy(sev.at[wid], ev_v)
    pltpu.sync_copy(sep.at[wid], ep_v)
    pltpu.sync_copy(srv.at[wid], rv_v)
    pltpu.sync_copy(srp.at[wid], rp_v)
    lanes = lax.broadcasted_iota(jnp.int32, (_LANES,), 0)
    zero16 = jnp.zeros((_LANES,), jnp.int32)

    def wait_pref(tab):
        pltpu.make_async_copy(tab.at[:, pl.ds(0, _W)],
                              bbuf.at[0], psem).wait()

    def stream(tab, tailtab, vals_v, pos_v, nhits):
        pltpu.sync_copy(tailtab, tail_v)

        def one_hit(v, p, k, state):
            cur, pref = state
            istail = v >= _TAIL
            blk = jnp.where(istail, cur, v // _W)

            @pl.when(blk != cur)
            def _():
                @pl.when(pref >= 0)
                def _():
                    wait_pref(tab)

                @pl.when(pref != blk)
                def _():
                    pltpu.sync_copy(tab.at[:, pl.ds(blk * _W, _W)],
                                    bbuf.at[blk % 2])

                @pl.when(blk + 1 < _NBLK)
                def _():
                    pltpu.async_copy(tab.at[:, pl.ds((blk + 1) * _W, _W)],
                                     bbuf.at[(blk + 1) % 2], psem)

            ncur = jnp.where(blk != cur, blk, cur)
            npref = jnp.where(blk != cur,
                              jnp.where(blk + 1 < _NBLK, blk + 1, -1),
                              pref)
            @pl.when(istail)
            def _():
                colt = zero16 + jnp.maximum(v - _TAIL, 0)
                for c in range(_DIM // _LANES):
                    dvec = c * _LANES + lanes
                    ering[k, pl.ds(c * _LANES, _LANES)] = (
                        plsc.load_gather(tail_v, [dvec, colt]))

            @pl.when(jnp.logical_not(istail))
            def _():
                par16 = zero16 + ncur % 2
                colm = zero16 + jnp.minimum(v - ncur * _W, _W - 1)
                for c in range(_DIM // _LANES):
                    dvec = c * _LANES + lanes
                    ering[k, pl.ds(c * _LANES, _LANES)] = (
                        plsc.load_gather(bbuf, [par16, dvec, colm]))

            pltpu.async_copy(ering.at[k],
                             scratch_out.at[pl.ds(p * _DIM, _DIM)], sem)
            return ncur, npref

        def ring_group(g, state):
            vv = vals_v[pl.ds(g * _RING, _LANES)]
            pv = pos_v[pl.ds(g * _RING, _LANES)]
            for j in range(_RING):
                state = one_hit(vv[j], pv[j], j, state)
            for j in range(_RING):
                pltpu.make_async_copy(ering.at[0],
                                      scratch_out.at[pl.ds(0, _DIM)],
                                      sem).wait()
            return state

        _, pref = lax.fori_loop(0, nhits // _RING, ring_group,
                                (jnp.int32(-1), jnp.int32(-1)))

        @pl.when(pref >= 0)
        def _():
            wait_pref(tab)

    stream(entT, tle, ev_v, ep_v, _EHITS)
    stream(relT, tlr, rv_v, rp_v, _RHITS)


_scan = functools.partial(
    pl.kernel,
    out_type=jax.ShapeDtypeStruct((_NSLOT * _DIM,), jnp.float32),
    mesh=plsc.VectorSubcoreMesh(core_axis_name="c", subcore_axis_name="s"),
    scratch_types=[
        pltpu.VMEM((_EHITS + _LANES,), jnp.int32),
        pltpu.VMEM((_EHITS + _LANES,), jnp.int32),
        pltpu.VMEM((_RHITS + _LANES,), jnp.int32),
        pltpu.VMEM((_RHITS + _LANES,), jnp.int32),
        pltpu.VMEM((2, _DIM, _W), jnp.float32),
        pltpu.VMEM((_DIM, _NTAIL), jnp.float32),
        pltpu.VMEM((_RING, _DIM), jnp.float32),
        pltpu.SemaphoreType.DMA,
        pltpu.SemaphoreType.DMA,
    ],
    compiler_params=pltpu.CompilerParams(
        needs_layout_passes=False, use_tc_tiling_on_sc=True),
)(_scan_body)


def _compute_body(scr, out_hbm, hbuf, rbuf, tbuf, out_v, sem):
    wid = lax.axis_index("s") * _NC + lax.axis_index("c")
    base = wid * _BPW
    c1 = pltpu.async_copy(scr.at[pl.ds(base, _BPW)], hbuf, sem)
    c2 = pltpu.async_copy(scr.at[pl.ds(_BATCH + base, _BPW)], tbuf, sem)
    c3 = pltpu.async_copy(scr.at[pl.ds(2 * _BATCH + base, _BPW)], rbuf, sem)
    c1.wait()
    c2.wait()
    c3.wait()
    lanes = lax.broadcasted_iota(jnp.int32, (_LANES,), 0)

    zero16 = jnp.zeros((_LANES,), jnp.int32)

    def group(g, carry):
        rows = g * _LANES + lanes

        def dim_body(d, acc):
            dvec = zero16 + d
            h = plsc.load_gather(hbuf, [rows, dvec])
            r = plsc.load_gather(rbuf, [rows, dvec])
            t = plsc.load_gather(tbuf, [rows, dvec])
            return acc + jnp.abs(h + r - t)

        acc = lax.fori_loop(0, _DIM, dim_body, jnp.zeros((_LANES,), jnp.float32))
        out_v[pl.ds(g * _LANES, _LANES)] = 1.0 / (1.0 + jnp.exp(acc - _GAMMA))
        return carry

    lax.fori_loop(0, _BPW // _LANES, group, 0)
    pltpu.sync_copy(out_v, out_hbm.at[pl.ds(base, _BPW)])


_compute = functools.partial(
    pl.kernel,
    out_type=jax.ShapeDtypeStruct((_BATCH,), jnp.float32),
    mesh=plsc.VectorSubcoreMesh(core_axis_name="c", subcore_axis_name="s"),
    scratch_types=[
        pltpu.VMEM((_BPW, _DIM), jnp.float32),
        pltpu.VMEM((_BPW, _DIM), jnp.float32),
        pltpu.VMEM((_BPW, _DIM), jnp.float32),
        pltpu.VMEM((_BPW,), jnp.float32),
        pltpu.SemaphoreType.DMA,
    ],
    compiler_params=pltpu.CompilerParams(
        needs_layout_passes=False, use_tc_tiling_on_sc=False),
)(_compute_body)


def _pad16(a):
    return jnp.pad(a, ((0, 0), (0, _LANES)))


def kernel(x, emb_ent_real, emb_rel_real):
    hv = x[:, 0].astype(jnp.int32)
    rv = x[:, 1].astype(jnp.int32)
    ev = jnp.concatenate([hv, rv])
    evs, eo = lax.sort((ev, jnp.arange(2 * _BATCH, dtype=jnp.int32)),
                       num_keys=1)
    rvs, ro = lax.sort((rv, jnp.arange(2 * _BATCH, 3 * _BATCH,
                                       dtype=jnp.int32)), num_keys=1)
    sev = _pad16(evs.reshape(_NW, _EHITS))
    sep = _pad16(eo.reshape(_NW, _EHITS))
    srv = _pad16(rvs.reshape(_NW, _RHITS))
    srp = _pad16(ro.reshape(_NW, _RHITS))
    scratch = _scan(sev, sep, srv, srp,
                    emb_ent_real.T, emb_rel_real.T,
                    emb_ent_real[_TAIL:, :].T, emb_rel_real[_TAIL:, :].T)
    return _compute(scratch.reshape(_NSLOT, _DIM))


# speedup vs baseline: 3.9498x; 1.0109x over previous
import functools

import jax
import jax.numpy as jnp
from jax import lax
from jax.experimental import pallas as pl
from jax.experimental.pallas import tpu as pltpu
from jax.experimental.pallas import tpu_sc as plsc

_GAMMA = 12.0
_DIM = 64
_BATCH = 16384
_NUM = 1000000
_NC = 2
_NS = 16
_NW = _NC * _NS
_BPW = _BATCH // _NW
_EHITS = 2 * _BATCH // _NW
_RHITS = _BATCH // _NW
_LANES = 16
_W = 768
_NBLK = _NUM // _W
_TAIL = _NBLK * _W
_NTAIL = _NUM - _TAIL
_RING = 8
_NSLOT = 3 * _BATCH


def _scan_body(sev, sep, srv, srp, entT, relT, tle, tlr, scratch_out,
               ev_v, ep_v, rv_v, rp_v, bbuf, tail_v, ering, sem, psem):
    wid = lax.axis_index("s") * _NC + lax.axis_index("c")
    pltpu.sync_copy(sev.at[wid], ev_v)
    pltpu.sync_copy(sep.at[wid], ep_v)
    pltpu.sync_copy(srv.at[wid], rv_v)
    pltpu.sync_copy(srp.at[wid], rp_v)
    lanes = lax.broadcasted_iota(jnp.int32, (_LANES,), 0)
    zero16 = jnp.zeros((_LANES,), jnp.int32)

    def wait_pref(tab):
        pltpu.make_async_copy(tab.at[:, pl.ds(0, _W)],
                              bbuf.at[0], psem).wait()

    def stream(tab, tailtab, vals_v, pos_v, nhits):
        pltpu.sync_copy(tailtab, tail_v)

        def one_hit(v, p, k, state):
            cur, pref = state
            istail = v >= _TAIL
            blk = jnp.where(istail, cur, v // _W)

            @pl.when(blk != cur)
            def _():
                @pl.when(pref >= 0)
                def _():
                    wait_pref(tab)

                @pl.when(pref != blk)
                def _():
                    pltpu.sync_copy(tab.at[:, pl.ds(blk * _W, _W)],
                                    bbuf.at[blk % 2])

                @pl.when(blk + 1 < _NBLK)
                def _():
                    pltpu.async_copy(tab.at[:, pl.ds((blk + 1) * _W, _W)],
                                     bbuf.at[(blk + 1) % 2], psem)

            ncur = jnp.where(blk != cur, blk, cur)
            npref = jnp.where(blk != cur,
                              jnp.where(blk + 1 < _NBLK, blk + 1, -1),
                              pref)
            @pl.when(istail)
            def _():
                colt = zero16 + jnp.maximum(v - _TAIL, 0)
                for c in range(_DIM // _LANES):
                    dvec = c * _LANES + lanes
                    ering[k, pl.ds(c * _LANES, _LANES)] = (
                        plsc.load_gather(tail_v, [dvec, colt]))

            @pl.when(jnp.logical_not(istail))
            def _():
                par16 = zero16 + ncur % 2
                colm = zero16 + jnp.minimum(v - ncur * _W, _W - 1)
                for c in range(_DIM // _LANES):
                    dvec = c * _LANES + lanes
                    ering[k, pl.ds(c * _LANES, _LANES)] = (
                        plsc.load_gather(bbuf, [par16, dvec, colm]))

            pltpu.async_copy(ering.at[k],
                             scratch_out.at[pl.ds(p * _DIM, _DIM)], sem)
            return ncur, npref

        def ring_group(g, state):
            vv = vals_v[pl.ds(g * _RING, _LANES)]
            pv = pos_v[pl.ds(g * _RING, _LANES)]
            for j in range(_RING):
                state = one_hit(vv[j], pv[j], j, state)
            for j in range(_RING):
                pltpu.make_async_copy(ering.at[0],
                                      scratch_out.at[pl.ds(0, _DIM)],
                                      sem).wait()
            return state

        _, pref = lax.fori_loop(0, nhits // _RING, ring_group,
                                (jnp.int32(-1), jnp.int32(-1)))

        @pl.when(pref >= 0)
        def _():
            wait_pref(tab)

    stream(entT, tle, ev_v, ep_v, _EHITS)
    stream(relT, tlr, rv_v, rp_v, _RHITS)


_scan = functools.partial(
    pl.kernel,
    out_type=jax.ShapeDtypeStruct((_NSLOT * _DIM,), jnp.float32),
    mesh=plsc.VectorSubcoreMesh(core_axis_name="c", subcore_axis_name="s"),
    scratch_types=[
        pltpu.VMEM((_EHITS + _LANES,), jnp.int32),
        pltpu.VMEM((_EHITS + _LANES,), jnp.int32),
        pltpu.VMEM((_RHITS + _LANES,), jnp.int32),
        pltpu.VMEM((_RHITS + _LANES,), jnp.int32),
        pltpu.VMEM((2, _DIM, _W), jnp.float32),
        pltpu.VMEM((_DIM, _NTAIL), jnp.float32),
        pltpu.VMEM((_RING, _DIM), jnp.float32),
        pltpu.SemaphoreType.DMA,
        pltpu.SemaphoreType.DMA,
    ],
    compiler_params=pltpu.CompilerParams(
        needs_layout_passes=False, use_tc_tiling_on_sc=True),
)(_scan_body)


def _compute_body(scr, out_hbm, hbuf, rbuf, tbuf, out_v, sem):
    wid = lax.axis_index("s") * _NC + lax.axis_index("c")
    base = wid * _BPW
    c1 = pltpu.async_copy(scr.at[pl.ds(base, _BPW)], hbuf, sem)
    c2 = pltpu.async_copy(scr.at[pl.ds(_BATCH + base, _BPW)], tbuf, sem)
    c3 = pltpu.async_copy(scr.at[pl.ds(2 * _BATCH + base, _BPW)], rbuf, sem)
    c1.wait()
    c2.wait()
    c3.wait()
    lanes = lax.broadcasted_iota(jnp.int32, (_LANES,), 0)

    zero16 = jnp.zeros((_LANES,), jnp.int32)

    def group(g, carry):
        rows = g * _LANES + lanes

        def dim_body(d, acc):
            dvec = zero16 + d
            h = plsc.load_gather(hbuf, [rows, dvec])
            r = plsc.load_gather(rbuf, [rows, dvec])
            t = plsc.load_gather(tbuf, [rows, dvec])
            return acc + jnp.abs(h + r - t)

        acc = lax.fori_loop(0, _DIM, dim_body,
                            jnp.zeros((_LANES,), jnp.float32), unroll=8)
        out_v[pl.ds(g * _LANES, _LANES)] = 1.0 / (1.0 + jnp.exp(acc - _GAMMA))
        return carry

    lax.fori_loop(0, _BPW // _LANES, group, 0)
    pltpu.sync_copy(out_v, out_hbm.at[pl.ds(base, _BPW)])


_compute = functools.partial(
    pl.kernel,
    out_type=jax.ShapeDtypeStruct((_BATCH,), jnp.float32),
    mesh=plsc.VectorSubcoreMesh(core_axis_name="c", subcore_axis_name="s"),
    scratch_types=[
        pltpu.VMEM((_BPW, _DIM), jnp.float32),
        pltpu.VMEM((_BPW, _DIM), jnp.float32),
        pltpu.VMEM((_BPW, _DIM), jnp.float32),
        pltpu.VMEM((_BPW,), jnp.float32),
        pltpu.SemaphoreType.DMA,
    ],
    compiler_params=pltpu.CompilerParams(
        needs_layout_passes=False, use_tc_tiling_on_sc=False),
)(_compute_body)


def _pad16(a):
    return jnp.pad(a, ((0, 0), (0, _LANES)))


def kernel(x, emb_ent_real, emb_rel_real):
    hv = x[:, 0].astype(jnp.int32)
    rv = x[:, 1].astype(jnp.int32)
    ev = jnp.concatenate([hv, rv])
    evs, eo = lax.sort((ev, jnp.arange(2 * _BATCH, dtype=jnp.int32)),
                       num_keys=1)
    rvs, ro = lax.sort((rv, jnp.arange(2 * _BATCH, 3 * _BATCH,
                                       dtype=jnp.int32)), num_keys=1)
    sev = _pad16(evs.reshape(_NW, _EHITS))
    sep = _pad16(eo.reshape(_NW, _EHITS))
    srv = _pad16(rvs.reshape(_NW, _RHITS))
    srp = _pad16(ro.reshape(_NW, _RHITS))
    scratch = _scan(sev, sep, srv, srp,
                    emb_ent_real.T, emb_rel_real.T,
                    emb_ent_real[_TAIL:, :].T, emb_rel_real[_TAIL:, :].T)
    return _compute(scratch.reshape(_NSLOT, _DIM))


# speedup vs baseline: 3.9656x; 1.0040x over previous
import functools

import jax
import jax.numpy as jnp
from jax import lax
from jax.experimental import pallas as pl
from jax.experimental.pallas import tpu as pltpu
from jax.experimental.pallas import tpu_sc as plsc

_GAMMA = 12.0
_DIM = 64
_BATCH = 16384
_NUM = 1000000
_NC = 2
_NS = 16
_NW = _NC * _NS
_BPW = _BATCH // _NW
_EHITS = 2 * _BATCH // _NW
_RHITS = _BATCH // _NW
_LANES = 16
_W = 768
_NBLK = _NUM // _W
_TAIL = _NBLK * _W
_NTAIL = _NUM - _TAIL
_RING = 8
_NSLOT = 3 * _BATCH


def _scan_body(svals, spos, tab, tailtab, scratch_out,
               vals_v, pos_v, bbuf, tail_v, ering, sem, psem, *, nhits):
    wid = lax.axis_index("s") * _NC + lax.axis_index("c")
    pltpu.sync_copy(svals.at[wid], vals_v)
    pltpu.sync_copy(spos.at[wid], pos_v)
    lanes = lax.broadcasted_iota(jnp.int32, (_LANES,), 0)
    zero16 = jnp.zeros((_LANES,), jnp.int32)

    def wait_pref(tab):
        pltpu.make_async_copy(tab.at[:, pl.ds(0, _W)],
                              bbuf.at[0], psem).wait()

    if True:
        pltpu.sync_copy(tailtab, tail_v)

        def one_hit(v, p, k, state):
            cur, pref = state
            istail = v >= _TAIL
            blk = jnp.where(istail, cur, v // _W)

            @pl.when(blk != cur)
            def _():
                @pl.when(pref >= 0)
                def _():
                    wait_pref(tab)

                @pl.when(pref != blk)
                def _():
                    pltpu.sync_copy(tab.at[:, pl.ds(blk * _W, _W)],
                                    bbuf.at[blk % 2])

                @pl.when(blk + 1 < _NBLK)
                def _():
                    pltpu.async_copy(tab.at[:, pl.ds((blk + 1) * _W, _W)],
                                     bbuf.at[(blk + 1) % 2], psem)

            ncur = jnp.where(blk != cur, blk, cur)
            npref = jnp.where(blk != cur,
                              jnp.where(blk + 1 < _NBLK, blk + 1, -1),
                              pref)
            @pl.when(istail)
            def _():
                colt = zero16 + jnp.maximum(v - _TAIL, 0)
                for c in range(_DIM // _LANES):
                    dvec = c * _LANES + lanes
                    ering[k, pl.ds(c * _LANES, _LANES)] = (
                        plsc.load_gather(tail_v, [dvec, colt]))

            @pl.when(jnp.logical_not(istail))
            def _():
                par16 = zero16 + ncur % 2
                colm = zero16 + jnp.minimum(v - ncur * _W, _W - 1)
                for c in range(_DIM // _LANES):
                    dvec = c * _LANES + lanes
                    ering[k, pl.ds(c * _LANES, _LANES)] = (
                        plsc.load_gather(bbuf, [par16, dvec, colm]))

            pltpu.async_copy(ering.at[k],
                             scratch_out.at[pl.ds(p * _DIM, _DIM)], sem)
            return ncur, npref

        def ring_group(g, state):
            vv = vals_v[pl.ds(g * _RING, _LANES)]
            pv = pos_v[pl.ds(g * _RING, _LANES)]
            for j in range(_RING):
                state = one_hit(vv[j], pv[j], j, state)
            for j in range(_RING):
                pltpu.make_async_copy(ering.at[0],
                                      scratch_out.at[pl.ds(0, _DIM)],
                                      sem).wait()
            return state

        _, pref = lax.fori_loop(0, nhits // _RING, ring_group,
                                (jnp.int32(-1), jnp.int32(-1)))

        @pl.when(pref >= 0)
        def _():
            wait_pref(tab)


def _make_scan(nhits):
    import functools as _ft
    return functools.partial(
        pl.kernel,
        out_type=jax.ShapeDtypeStruct((_NW * nhits * _DIM,), jnp.float32),
        mesh=plsc.VectorSubcoreMesh(core_axis_name="c", subcore_axis_name="s"),
        scratch_types=[
            pltpu.VMEM((nhits + _LANES,), jnp.int32),
            pltpu.VMEM((nhits + _LANES,), jnp.int32),
            pltpu.VMEM((2, _DIM, _W), jnp.float32),
            pltpu.VMEM((_DIM, _NTAIL), jnp.float32),
            pltpu.VMEM((_RING, _DIM), jnp.float32),
            pltpu.SemaphoreType.DMA,
            pltpu.SemaphoreType.DMA,
        ],
        compiler_params=pltpu.CompilerParams(
            needs_layout_passes=False, use_tc_tiling_on_sc=True),
    )(_ft.partial(_scan_body, nhits=nhits))


_scan_ent = _make_scan(_EHITS)
_scan_rel = _make_scan(_RHITS)


def _compute_body(scr_e, scr_r, out_hbm, hbuf, rbuf, tbuf, out_v, sem):
    wid = lax.axis_index("s") * _NC + lax.axis_index("c")
    base = wid * _BPW
    c1 = pltpu.async_copy(scr_e.at[pl.ds(base, _BPW)], hbuf, sem)
    c2 = pltpu.async_copy(scr_e.at[pl.ds(_BATCH + base, _BPW)], tbuf, sem)
    c3 = pltpu.async_copy(scr_r.at[pl.ds(base, _BPW)], rbuf, sem)
    c1.wait()
    c2.wait()
    c3.wait()
    lanes = lax.broadcasted_iota(jnp.int32, (_LANES,), 0)

    zero16 = jnp.zeros((_LANES,), jnp.int32)

    def group(g, carry):
        rows = g * _LANES + lanes

        def dim_body(d, acc):
            dvec = zero16 + d
            h = plsc.load_gather(hbuf, [rows, dvec])
            r = plsc.load_gather(rbuf, [rows, dvec])
            t = plsc.load_gather(tbuf, [rows, dvec])
            return acc + jnp.abs(h + r - t)

        acc = lax.fori_loop(0, _DIM, dim_body,
                            jnp.zeros((_LANES,), jnp.float32), unroll=8)
        out_v[pl.ds(g * _LANES, _LANES)] = 1.0 / (1.0 + jnp.exp(acc - _GAMMA))
        return carry

    lax.fori_loop(0, _BPW // _LANES, group, 0)
    pltpu.sync_copy(out_v, out_hbm.at[pl.ds(base, _BPW)])


_compute = functools.partial(
    pl.kernel,
    out_type=jax.ShapeDtypeStruct((_BATCH,), jnp.float32),
    mesh=plsc.VectorSubcoreMesh(core_axis_name="c", subcore_axis_name="s"),
    scratch_types=[
        pltpu.VMEM((_BPW, _DIM), jnp.float32),
        pltpu.VMEM((_BPW, _DIM), jnp.float32),
        pltpu.VMEM((_BPW, _DIM), jnp.float32),
        pltpu.VMEM((_BPW,), jnp.float32),
        pltpu.SemaphoreType.DMA,
    ],
    compiler_params=pltpu.CompilerParams(
        needs_layout_passes=False, use_tc_tiling_on_sc=False),
)(_compute_body)


def _pad16(a):
    return jnp.pad(a, ((0, 0), (0, _LANES)))


def kernel(x, emb_ent_real, emb_rel_real):
    hv = x[:, 0].astype(jnp.int32)
    rv = x[:, 1].astype(jnp.int32)
    ev = jnp.concatenate([hv, rv])
    evs, eo = lax.sort((ev, jnp.arange(2 * _BATCH, dtype=jnp.int32)),
                       num_keys=1)
    sev = _pad16(evs.reshape(_NW, _EHITS))
    sep = _pad16(eo.reshape(_NW, _EHITS))
    scr_e = _scan_ent(sev, sep, emb_ent_real.T, emb_ent_real[_TAIL:, :].T)
    rvs, ro = lax.sort((rv, jnp.arange(_BATCH, dtype=jnp.int32)), num_keys=1)
    srv = _pad16(rvs.reshape(_NW, _RHITS))
    srp = _pad16(ro.reshape(_NW, _RHITS))
    scr_r = _scan_rel(srv, srp, emb_rel_real.T, emb_rel_real[_TAIL:, :].T)
    return _compute(scr_e.reshape(2 * _BATCH, _DIM),
                    scr_r.reshape(_BATCH, _DIM))
